# baseline (speedup 1.0000x reference)
import functools

import jax
import jax.numpy as jnp
from jax import lax
from jax.experimental import pallas as pl
from jax.experimental.pallas import tpu as pltpu
from jax.experimental.pallas import tpu_sc as plsc

N = 10000
E = 320000
D = 128

NC = 2
NS = 16
NW = NC * NS
EPW = E // NW
CH = 100
NCH = EPW // CH
NP = 10112
RPS = NP // NS
CW = 16


def _sc_agg(x, src, dst, zf, zc, ones_rows):
  mesh = plsc.VectorSubcoreMesh(core_axis_name="c", subcore_axis_name="s")

  @functools.partial(
      pl.kernel,
      out_type=(jax.ShapeDtypeStruct((NC, NP, D), jnp.float32),
                jax.ShapeDtypeStruct((NC, NP, CW), jnp.float32)),
      mesh=mesh,
      scratch_types=[
          pltpu.VMEM((4, CH), jnp.int32),
          pltpu.VMEM((4, CH), jnp.int32),
          pltpu.VMEM((CH, D), jnp.float32),
          pltpu.VMEM((CH, D), jnp.float32),
          pltpu.VMEM((CH, CW), jnp.float32),
          pltpu.VMEM_SHARED((NP, D), jnp.float32),
          pltpu.VMEM_SHARED((NP, CW), jnp.float32),
          [pltpu.SemaphoreType.DMA] * 8,
      ],
      compiler_params=pltpu.CompilerParams(use_tc_tiling_on_sc=False),
  )
  def k(x_hbm, src_hbm, dst_hbm, zf_hbm, zc_hbm, ones_hbm,
        outf_hbm, outc_hbm, src_v, dst_v, buf0, buf1, ones_v,
        facc, cacc, sems):
    (gs0, gs1, ss0, ss1, cs0, cs1, is_s, is_d) = sems
    cid = lax.axis_index("c")
    sid = lax.axis_index("s")
    wid = sid * NC + cid

    pltpu.sync_copy(zf_hbm.at[pl.ds(sid * RPS, RPS)],
                    facc.at[pl.ds(sid * RPS, RPS)])
    pltpu.sync_copy(zc_hbm.at[pl.ds(sid * RPS, RPS)],
                    cacc.at[pl.ds(sid * RPS, RPS)])
    pltpu.sync_copy(ones_hbm, ones_v)
    pltpu.sync_copy(src_hbm.at[wid, pl.ds(0, 2)], src_v.at[pl.ds(0, 2)])
    pltpu.sync_copy(dst_hbm.at[wid, pl.ds(0, 2)], dst_v.at[pl.ds(0, 2)])
    plsc.subcore_barrier()

    def iload(blk, rows):
      pltpu.async_copy(src_hbm.at[wid, pl.ds(2 * blk, 2)],
                       src_v.at[pl.ds(rows, 2)], is_s)
      pltpu.async_copy(dst_hbm.at[wid, pl.ds(2 * blk, 2)],
                       dst_v.at[pl.ds(rows, 2)], is_d)

    def iwait(blk, rows):
      pltpu.make_async_copy(src_hbm.at[wid, pl.ds(2 * blk, 2)],
                            src_v.at[pl.ds(rows, 2)], is_s).wait()
      pltpu.make_async_copy(dst_hbm.at[wid, pl.ds(2 * blk, 2)],
                            dst_v.at[pl.ds(rows, 2)], is_d).wait()

    def gather(row, buf, sem):
      pltpu.async_copy(x_hbm.at[src_v.at[row]], buf, sem)

    def gwait(row, buf, sem):
      pltpu.make_async_copy(x_hbm.at[src_v.at[row]], buf, sem).wait()

    def sfire(row, buf, fsem, csem):
      pltpu.async_copy(buf, facc.at[dst_v.at[row]], fsem, add=True)
      pltpu.async_copy(ones_v, cacc.at[dst_v.at[row]], csem, add=True)

    def swait(row, buf, fsem, csem):
      pltpu.make_async_copy(buf, facc.at[dst_v.at[row]], fsem).wait()
      pltpu.make_async_copy(ones_v, cacc.at[dst_v.at[row]], csem).wait()

    gather(0, buf0, gs0)
    gather(1, buf1, gs1)
    iload(1, 2)

    def body(t, carry):
      a = 2 * lax.rem(t, 2)
      b = 2 - a
      gwait(a, buf0, gs0)
      sfire(a, buf0, ss0, cs0)
      gwait(a + 1, buf1, gs1)
      sfire(a + 1, buf1, ss1, cs1)
      iwait(t + 1, b)
      swait(a, buf0, ss0, cs0)
      gather(b, buf0, gs0)
      swait(a + 1, buf1, ss1, cs1)
      gather(b + 1, buf1, gs1)
      iload(t + 2, a)
      return carry

    lax.fori_loop(0, NCH // 2 - 2, body, 0)
    gwait(0, buf0, gs0)
    sfire(0, buf0, ss0, cs0)
    gwait(1, buf1, gs1)
    sfire(1, buf1, ss1, cs1)
    iwait(NCH // 2 - 1, 2)
    swait(0, buf0, ss0, cs0)
    gather(2, buf0, gs0)
    swait(1, buf1, ss1, cs1)
    gather(3, buf1, gs1)
    gwait(2, buf0, gs0)
    sfire(2, buf0, ss0, cs0)
    gwait(3, buf1, gs1)
    sfire(3, buf1, ss1, cs1)
    swait(2, buf0, ss0, cs0)
    swait(3, buf1, ss1, cs1)

    plsc.subcore_barrier()
    pltpu.sync_copy(facc.at[pl.ds(sid * RPS, RPS)],
                    outf_hbm.at[cid, pl.ds(sid * RPS, RPS)])
    pltpu.sync_copy(cacc.at[pl.ds(sid * RPS, RPS)],
                    outc_hbm.at[cid, pl.ds(sid * RPS, RPS)])

  return k(x, src, dst, zf, zc, ones_rows)


def _tc_dense(acc, cacc, w_t, gamma, beta):
  def body(acc_ref, c_ref, w_ref, g_ref, b_ref, out_ref):
    s = acc_ref[0, :N] + acc_ref[1, :N]
    cnt = c_ref[0, :N, 0:1] + c_ref[1, :N, 0:1]
    mean = s / jnp.maximum(cnt, 1.0)
    h = jnp.dot(mean, w_ref[...], preferred_element_type=jnp.float32)
    h = jnp.where(h > 0, h, jnp.exp(jnp.minimum(h, 0.0)) - 1.0)
    mu = jnp.mean(h, axis=0, keepdims=True)
    xc = h - mu
    var = jnp.mean(xc * xc, axis=0, keepdims=True)
    out_ref[...] = g_ref[...] * (xc * lax.rsqrt(var + 1e-5)) + b_ref[...]

  return pl.pallas_call(
      body,
      out_shape=jax.ShapeDtypeStruct((N, D), jnp.float32),
  )(acc, cacc, w_t, gamma, beta)


def kernel(x, edge_index, W, gamma, beta):
  src = edge_index[0].reshape(NW, NCH, CH)
  dst = edge_index[1].reshape(NW, NCH, CH)
  zf = jnp.zeros((NP, D), jnp.float32)
  zc = jnp.zeros((NP, CW), jnp.float32)
  ones_rows = jnp.ones((CH, CW), jnp.float32)
  acc, cacc = _sc_agg(x, src, dst, zf, zc, ones_rows)
  return _tc_dense(acc, cacc, W.T, gamma.reshape(1, D), beta.reshape(1, D))
